# baseline (speedup 1.0000x reference)
import jax
import jax.numpy as jnp
from jax import lax
from jax.experimental import pallas as pl
from jax.experimental.pallas import tpu as pltpu
from jax.experimental.pallas import tpu_sc as plsc

_B = 16384
_F = 26
_V = 100000
_D = 32
_GRP = _B // (16 * 8)


def _body(x_hbm, tab_hbm, out_hbm, row_v, buf_v, rsem, isem, osem):
    d = lax.axis_index("s") * 2 + lax.axis_index("c")

    row_cp = pltpu.async_copy(tab_hbm.at[0, d], row_v, rsem)
    out_cp = None
    for f in range(_F):
        if out_cp is not None:
            out_cp.wait()
        idx_cp = pltpu.async_copy(x_hbm.at[f], buf_v, isem)
        row_cp.wait()
        idx_cp.wait()

        @plsc.parallel_loop(0, _B, step=16, unroll=8)
        def gather16(i):
            sl = pl.ds(i, 16)
            iv = plsc.bitcast(buf_v[sl], jnp.int32)
            buf_v[sl] = plsc.load_gather(row_v, [iv])

        if f + 1 < _F:
            row_cp = pltpu.async_copy(tab_hbm.at[f + 1, d], row_v, rsem)
        out_cp = pltpu.async_copy(buf_v, out_hbm.at[f, d], osem)
    out_cp.wait()


@jax.jit
def kernel(x_cat, tables):
    x_t = jax.lax.bitcast_convert_type(x_cat.T, jnp.float32)
    tab_t = jnp.transpose(tables, (0, 2, 1))
    mesh = plsc.VectorSubcoreMesh(core_axis_name="c", subcore_axis_name="s")
    out = pl.kernel(
        _body,
        mesh=mesh,
        out_type=jax.ShapeDtypeStruct((_F, _D, _B), jnp.float32),
        scratch_types=[
            pltpu.VMEM((_V,), jnp.float32),
            pltpu.VMEM((_B,), jnp.float32),
            pltpu.SemaphoreType.DMA,
            pltpu.SemaphoreType.DMA,
            pltpu.SemaphoreType.DMA,
        ],
        compiler_params=pltpu.CompilerParams(
            use_tc_tiling_on_sc=True, needs_layout_passes=False
        ),
    )(x_t, tab_t)
    return jnp.transpose(out, (2, 0, 1))


# speedup vs baseline: 1.0005x; 1.0005x over previous
import jax
import jax.numpy as jnp
from jax import lax
from jax.experimental import pallas as pl
from jax.experimental.pallas import tpu as pltpu
from jax.experimental.pallas import tpu_sc as plsc

_B = 16384
_F = 26
_V = 100000
_D = 32
_GRP = _B // (16 * 8)


def _body(x_hbm, tab_hbm, out_hbm, row_v, buf_v, rsem, isem, osem):
    d = lax.axis_index("s") * 2 + lax.axis_index("c")

    row_cp = pltpu.async_copy(tab_hbm.at[0, d], row_v, rsem)
    out_cp = None
    for f in range(_F):
        if out_cp is not None:
            out_cp.wait()
        idx_cp = pltpu.async_copy(x_hbm.at[f], buf_v, isem)
        row_cp.wait()
        idx_cp.wait()

        @plsc.parallel_loop(0, _B, step=16, unroll=16)
        def gather16(i):
            sl = pl.ds(i, 16)
            iv = plsc.bitcast(buf_v[sl], jnp.int32)
            buf_v[sl] = plsc.load_gather(row_v, [iv])

        if f + 1 < _F:
            row_cp = pltpu.async_copy(tab_hbm.at[f + 1, d], row_v, rsem)
        out_cp = pltpu.async_copy(buf_v, out_hbm.at[f, d], osem)
    out_cp.wait()


@jax.jit
def kernel(x_cat, tables):
    x_t = jax.lax.bitcast_convert_type(x_cat.T, jnp.float32)
    tab_t = jnp.transpose(tables, (0, 2, 1))
    mesh = plsc.VectorSubcoreMesh(core_axis_name="c", subcore_axis_name="s")
    out = pl.kernel(
        _body,
        mesh=mesh,
        out_type=jax.ShapeDtypeStruct((_F, _D, _B), jnp.float32),
        scratch_types=[
            pltpu.VMEM((_V,), jnp.float32),
            pltpu.VMEM((_B,), jnp.float32),
            pltpu.SemaphoreType.DMA,
            pltpu.SemaphoreType.DMA,
            pltpu.SemaphoreType.DMA,
        ],
        compiler_params=pltpu.CompilerParams(
            use_tc_tiling_on_sc=True, needs_layout_passes=False
        ),
    )(x_t, tab_t)
    return jnp.transpose(out, (2, 0, 1))


# speedup vs baseline: 1.0080x; 1.0075x over previous
import jax
import jax.numpy as jnp
from jax import lax
from jax.experimental import pallas as pl
from jax.experimental.pallas import tpu as pltpu
from jax.experimental.pallas import tpu_sc as plsc

_B = 16384
_F = 26
_V = 100000
_D = 32
_GRP = _B // (16 * 8)


def _body(x_hbm, tab_hbm, out_hbm, row_v, buf_v, rsem, osem):
    d = lax.axis_index("c") * 16 + lax.axis_index("s")

    row_cp = pltpu.async_copy(tab_hbm.at[0, d], row_v, rsem)
    out_cp = None
    for f in range(_F):
        if out_cp is not None:
            out_cp.wait()
        pltpu.sync_copy(x_hbm.at[f], buf_v)
        row_cp.wait()

        @plsc.parallel_loop(0, _B, step=16, unroll=8)
        def gather16(i):
            sl = pl.ds(i, 16)
            iv = plsc.bitcast(buf_v[sl], jnp.int32)
            buf_v[sl] = plsc.load_gather(row_v, [iv])

        if f + 1 < _F:
            row_cp = pltpu.async_copy(tab_hbm.at[f + 1, d], row_v, rsem)
        out_cp = pltpu.async_copy(buf_v, out_hbm.at[f, d], osem)
    out_cp.wait()


@jax.jit
def kernel(x_cat, tables):
    x_t = jax.lax.bitcast_convert_type(x_cat.T, jnp.float32)
    tab_t = jnp.transpose(tables, (0, 2, 1))
    mesh = plsc.VectorSubcoreMesh(core_axis_name="c", subcore_axis_name="s")
    out = pl.kernel(
        _body,
        mesh=mesh,
        out_type=jax.ShapeDtypeStruct((_F, _D, _B), jnp.float32),
        scratch_types=[
            pltpu.VMEM((_V,), jnp.float32),
            pltpu.VMEM((_B,), jnp.float32),
            pltpu.SemaphoreType.DMA,
            pltpu.SemaphoreType.DMA,
        ],
        compiler_params=pltpu.CompilerParams(
            use_tc_tiling_on_sc=True, needs_layout_passes=False
        ),
    )(x_t, tab_t)
    return jnp.transpose(out, (2, 0, 1))
